# baseline (speedup 1.0000x reference)
import functools

import jax
import jax.numpy as jnp
from jax import lax
from jax.experimental import pallas as pl
from jax.experimental.pallas import tpu as pltpu
from jax.experimental.pallas import tpu_sc as plsc

N_NODES = 10000
N_EDGES = 320000
F = 128
NC = 2
NS = 16
NW = NC * NS
EPW = N_EDGES // NW
CHUNK = 40
NCHUNK = EPW // CHUNK
NBUF = 8
NPAD = 10240
RPW = NPAD // NS
ZR = 32


def _sc_aggregate(x, ei_flat):
    mesh = plsc.VectorSubcoreMesh(core_axis_name="c", subcore_axis_name="s")

    @functools.partial(
        pl.kernel,
        mesh=mesh,
        out_type=[
            jax.ShapeDtypeStruct((NC * NPAD, F), jnp.float32),
            jax.ShapeDtypeStruct((NC * NPAD,), jnp.float32),
        ],
        scratch_types=(
            [pltpu.VMEM((CHUNK,), jnp.int32)] * NBUF
            + [pltpu.VMEM((CHUNK,), jnp.int32)] * NBUF
            + [pltpu.VMEM((CHUNK, F), jnp.float32)] * NBUF
            + [
                pltpu.VMEM((CHUNK,), jnp.float32),
                pltpu.VMEM((ZR, F), jnp.float32),
                pltpu.VMEM((RPW,), jnp.float32),
                pltpu.VMEM_SHARED((NPAD, F), jnp.float32),
                pltpu.VMEM_SHARED((NPAD,), jnp.float32),
            ]
            + [pltpu.SemaphoreType.DMA] * (3 * NBUF)
        ),
    )
    def agg(x_hbm, ei_hbm, sum_hbm, cnt_hbm, *rest):
        srcs = rest[0:NBUF]
        dsts = rest[NBUF:2 * NBUF]
        rows = rest[2 * NBUF:3 * NBUF]
        ones_v, zrow_v, zcnt_v, ssum, scnt = rest[3 * NBUF:3 * NBUF + 5]
        sems = rest[3 * NBUF + 5:]
        sem_i = sems[0:NBUF]
        sem_g = sems[NBUF:2 * NBUF]
        sem_s = sems[2 * NBUF:3 * NBUF]
        cid = lax.axis_index("c")
        sid = lax.axis_index("s")
        wid = sid * NC + cid

        zero16 = jnp.zeros((16,), jnp.float32)
        one16 = jnp.ones((16,), jnp.float32)

        def fill_zeros(i, carry):
            for j in range(F // 16):
                zrow_v[i, pl.ds(j * 16, 16)] = zero16
            return carry

        lax.fori_loop(0, ZR, fill_zeros, 0)

        def fill_ones(i, carry):
            ones_v[pl.ds(i * 16, 16)] = one16
            return carry

        lax.fori_loop(0, CHUNK // 16, fill_ones, 0)

        def fill_zcnt(i, carry):
            zcnt_v[pl.ds(i * 16, 16)] = zero16
            return carry

        lax.fori_loop(0, RPW // 16, fill_zcnt, 0)

        r0 = sid * RPW
        for q in range(RPW // ZR):
            pltpu.sync_copy(zrow_v, ssum.at[pl.ds(r0 + q * ZR, ZR)])
        pltpu.sync_copy(zcnt_v, scnt.at[pl.ds(r0, RPW)])

        plsc.subcore_barrier()

        ebase = wid * EPW

        def fire_idx(c, b):
            base = ebase + c * CHUNK
            pltpu.async_copy(ei_hbm.at[pl.ds(base, CHUNK)], srcs[b],
                             sem_i[b])
            pltpu.async_copy(ei_hbm.at[pl.ds(N_EDGES + base, CHUNK)], dsts[b],
                             sem_i[b])

        def wait_idx(b):
            pltpu.make_async_copy(ei_hbm.at[pl.ds(0, CHUNK)], srcs[b],
                                  sem_i[b]).wait()
            pltpu.make_async_copy(ei_hbm.at[pl.ds(0, CHUNK)], dsts[b],
                                  sem_i[b]).wait()

        def fire_gather(b):
            pltpu.async_copy(x_hbm.at[srcs[b]], rows[b], sem_g[b])

        def wait_gather(b):
            pltpu.make_async_copy(x_hbm.at[srcs[b]], rows[b],
                                  sem_g[b]).wait()

        def fire_scatter(b):
            pltpu.async_copy(rows[b], ssum.at[dsts[b]], sem_s[b], add=True)
            pltpu.async_copy(ones_v, scnt.at[dsts[b]], sem_s[b], add=True)

        def wait_scatter(b):
            pltpu.make_async_copy(rows[b], ssum.at[dsts[b]], sem_s[b]).wait()
            pltpu.make_async_copy(ones_v, scnt.at[dsts[b]], sem_s[b]).wait()

        def step(i, reuse):
            if reuse:
                wait_scatter(i % NBUF)
            fire_idx(i, i % NBUF)
            if i >= 1:
                wait_idx((i - 1) % NBUF)
                fire_gather((i - 1) % NBUF)
            if i >= 2:
                wait_gather((i - 2) % NBUF)
                fire_scatter((i - 2) % NBUF)

        for i in range(NBUF):
            step(i, False)

        K = (NCHUNK - NBUF) // NBUF

        def body(k, carry):
            c = NBUF * k
            for j in range(NBUF):
                wait_scatter(j)
                fire_idx(c + j, j)
                wait_idx((j - 1) % NBUF)
                fire_gather((j - 1) % NBUF)
                wait_gather((j - 2) % NBUF)
                fire_scatter((j - 2) % NBUF)
            return carry

        lax.fori_loop(1, K + 1, body, 0)

        for i in range(NBUF * (K + 1), NCHUNK):
            step(i, True)

        last = NCHUNK - 1
        wait_idx(last % NBUF)
        fire_gather(last % NBUF)
        wait_gather((last - 1) % NBUF)
        fire_scatter((last - 1) % NBUF)
        wait_gather(last % NBUF)
        fire_scatter(last % NBUF)
        for b in range(NBUF):
            wait_scatter(b)

        plsc.subcore_barrier()

        out_r0 = cid * NPAD + r0
        pltpu.sync_copy(ssum.at[pl.ds(r0, RPW)], sum_hbm.at[pl.ds(out_r0, RPW)])
        pltpu.sync_copy(scnt.at[pl.ds(r0, RPW)], cnt_hbm.at[pl.ds(out_r0, RPW)])

    return agg(x, ei_flat)


def _tc_combine(x, psum, pcnt, wl, wr, bias):
    R = 1024

    def body(x_ref, p0_ref, p1_ref, c0_ref, c1_ref, wl_ref, wr_ref, b_ref,
             o_ref):
        s = p0_ref[0] + p1_ref[0]
        cnt = c0_ref[0] + c1_ref[0]
        mean = s / jnp.maximum(cnt, 1.0)
        dn = (((1,), (1,)), ((), ()))
        acc = lax.dot_general(mean, wl_ref[...], dn,
                              preferred_element_type=jnp.float32)
        acc = acc + lax.dot_general(x_ref[...], wr_ref[...], dn,
                                    preferred_element_type=jnp.float32)
        o_ref[...] = jnp.maximum(acc + b_ref[...], 0.0)

    return pl.pallas_call(
        body,
        grid=(pl.cdiv(N_NODES, R),),
        in_specs=[
            pl.BlockSpec((R, F), lambda i: (i, 0)),
            pl.BlockSpec((1, R, F), lambda i: (0, i, 0)),
            pl.BlockSpec((1, R, F), lambda i: (1, i, 0)),
            pl.BlockSpec((1, R, 1), lambda i: (0, i, 0)),
            pl.BlockSpec((1, R, 1), lambda i: (1, i, 0)),
            pl.BlockSpec((F, F), lambda i: (0, 0)),
            pl.BlockSpec((F, F), lambda i: (0, 0)),
            pl.BlockSpec((1, F), lambda i: (0, 0)),
        ],
        out_specs=pl.BlockSpec((R, F), lambda i: (i, 0)),
        out_shape=jax.ShapeDtypeStruct((N_NODES, F), jnp.float32),
    )(x, psum, psum, pcnt, pcnt, wl, wr, bias)


def kernel(x, edge_index, W_l, b_l, W_r, b_r):
    ei_flat = edge_index.astype(jnp.int32).reshape(2 * N_EDGES)
    psum, pcnt = _sc_aggregate(x, ei_flat)
    psum3 = psum.reshape(NC, NPAD, F)
    pcnt3 = pcnt.reshape(NC, NPAD, 1)
    bias = (b_l + b_r).reshape(1, F)
    return _tc_combine(x, psum3, pcnt3, W_l, W_r, bias)


# speedup vs baseline: 1.2412x; 1.2412x over previous
import functools

import jax
import jax.numpy as jnp
from jax import lax
from jax.experimental import pallas as pl
from jax.experimental.pallas import tpu as pltpu
from jax.experimental.pallas import tpu_sc as plsc

N_NODES = 10000
N_EDGES = 320000
F = 128
NC = 2
NS = 16
NW = NC * NS
EPW = N_EDGES // NW
CHUNK = 80
NCHUNK = EPW // CHUNK
NBUF = 4
NPAD = 10240
RPW = NPAD // NS
ZR = 32


def _sc_aggregate(x, ei_flat):
    mesh = plsc.VectorSubcoreMesh(core_axis_name="c", subcore_axis_name="s")

    @functools.partial(
        pl.kernel,
        mesh=mesh,
        out_type=[
            jax.ShapeDtypeStruct((NC * NPAD, F), jnp.float32),
            jax.ShapeDtypeStruct((NC * NPAD,), jnp.float32),
        ],
        scratch_types=(
            [pltpu.VMEM((CHUNK,), jnp.int32)] * NBUF
            + [pltpu.VMEM((CHUNK,), jnp.int32)] * NBUF
            + [pltpu.VMEM((CHUNK, F), jnp.float32)] * NBUF
            + [
                pltpu.VMEM((CHUNK,), jnp.float32),
                pltpu.VMEM((ZR, F), jnp.float32),
                pltpu.VMEM((RPW,), jnp.float32),
                pltpu.VMEM_SHARED((NPAD, F), jnp.float32),
                pltpu.VMEM_SHARED((NPAD,), jnp.float32),
            ]
            + [pltpu.SemaphoreType.DMA] * (3 * NBUF)
        ),
    )
    def agg(x_hbm, ei_hbm, sum_hbm, cnt_hbm, *rest):
        srcs = rest[0:NBUF]
        dsts = rest[NBUF:2 * NBUF]
        rows = rest[2 * NBUF:3 * NBUF]
        ones_v, zrow_v, zcnt_v, ssum, scnt = rest[3 * NBUF:3 * NBUF + 5]
        sems = rest[3 * NBUF + 5:]
        sem_i = sems[0:NBUF]
        sem_g = sems[NBUF:2 * NBUF]
        sem_s = sems[2 * NBUF:3 * NBUF]
        cid = lax.axis_index("c")
        sid = lax.axis_index("s")
        wid = sid * NC + cid

        zero16 = jnp.zeros((16,), jnp.float32)
        one16 = jnp.ones((16,), jnp.float32)

        def fill_zeros(i, carry):
            for j in range(F // 16):
                zrow_v[i, pl.ds(j * 16, 16)] = zero16
            return carry

        lax.fori_loop(0, ZR, fill_zeros, 0)

        def fill_ones(i, carry):
            ones_v[pl.ds(i * 16, 16)] = one16
            return carry

        lax.fori_loop(0, CHUNK // 16, fill_ones, 0)

        def fill_zcnt(i, carry):
            zcnt_v[pl.ds(i * 16, 16)] = zero16
            return carry

        lax.fori_loop(0, RPW // 16, fill_zcnt, 0)

        r0 = sid * RPW
        for q in range(RPW // ZR):
            pltpu.sync_copy(zrow_v, ssum.at[pl.ds(r0 + q * ZR, ZR)])
        pltpu.sync_copy(zcnt_v, scnt.at[pl.ds(r0, RPW)])

        plsc.subcore_barrier()

        ebase = wid * EPW

        def fire_idx(c, b):
            base = ebase + c * CHUNK
            pltpu.async_copy(ei_hbm.at[pl.ds(base, CHUNK)], srcs[b],
                             sem_i[b])
            pltpu.async_copy(ei_hbm.at[pl.ds(N_EDGES + base, CHUNK)], dsts[b],
                             sem_i[b])

        def wait_idx(b):
            pltpu.make_async_copy(ei_hbm.at[pl.ds(0, CHUNK)], srcs[b],
                                  sem_i[b]).wait()
            pltpu.make_async_copy(ei_hbm.at[pl.ds(0, CHUNK)], dsts[b],
                                  sem_i[b]).wait()

        def fire_gather(b):
            pltpu.async_copy(x_hbm.at[srcs[b]], rows[b], sem_g[b])

        def wait_gather(b):
            pltpu.make_async_copy(x_hbm.at[srcs[b]], rows[b],
                                  sem_g[b]).wait()

        def fire_scatter(b):
            pltpu.async_copy(rows[b], ssum.at[dsts[b]], sem_s[b], add=True)
            pltpu.async_copy(ones_v, scnt.at[dsts[b]], sem_s[b], add=True)

        def wait_scatter(b):
            pltpu.make_async_copy(rows[b], ssum.at[dsts[b]], sem_s[b]).wait()
            pltpu.make_async_copy(ones_v, scnt.at[dsts[b]], sem_s[b]).wait()

        def step(i, reuse):
            if reuse:
                wait_scatter(i % NBUF)
            fire_idx(i, i % NBUF)
            if i >= 1:
                wait_idx((i - 1) % NBUF)
                fire_gather((i - 1) % NBUF)
            if i >= 2:
                wait_gather((i - 2) % NBUF)
                fire_scatter((i - 2) % NBUF)

        for i in range(NBUF):
            step(i, False)

        K = (NCHUNK - NBUF) // NBUF

        def body(k, carry):
            c = NBUF * k
            for j in range(NBUF):
                wait_scatter(j)
                fire_idx(c + j, j)
                wait_idx((j - 1) % NBUF)
                fire_gather((j - 1) % NBUF)
                wait_gather((j - 2) % NBUF)
                fire_scatter((j - 2) % NBUF)
            return carry

        lax.fori_loop(1, K + 1, body, 0)

        for i in range(NBUF * (K + 1), NCHUNK):
            step(i, True)

        last = NCHUNK - 1
        wait_idx(last % NBUF)
        fire_gather(last % NBUF)
        wait_gather((last - 1) % NBUF)
        fire_scatter((last - 1) % NBUF)
        wait_gather(last % NBUF)
        fire_scatter(last % NBUF)
        for b in range(NBUF):
            wait_scatter(b)

        plsc.subcore_barrier()

        out_r0 = cid * NPAD + r0
        pltpu.sync_copy(ssum.at[pl.ds(r0, RPW)], sum_hbm.at[pl.ds(out_r0, RPW)])
        pltpu.sync_copy(scnt.at[pl.ds(r0, RPW)], cnt_hbm.at[pl.ds(out_r0, RPW)])

    return agg(x, ei_flat)


def _tc_combine(x, psum, pcnt, wl, wr, bias):
    R = 1024

    def body(x_ref, p0_ref, p1_ref, c0_ref, c1_ref, wl_ref, wr_ref, b_ref,
             o_ref):
        s = p0_ref[0] + p1_ref[0]
        cnt = c0_ref[0] + c1_ref[0]
        mean = s / jnp.maximum(cnt, 1.0)
        dn = (((1,), (1,)), ((), ()))
        acc = lax.dot_general(mean, wl_ref[...], dn,
                              preferred_element_type=jnp.float32)
        acc = acc + lax.dot_general(x_ref[...], wr_ref[...], dn,
                                    preferred_element_type=jnp.float32)
        o_ref[...] = jnp.maximum(acc + b_ref[...], 0.0)

    return pl.pallas_call(
        body,
        grid=(pl.cdiv(N_NODES, R),),
        in_specs=[
            pl.BlockSpec((R, F), lambda i: (i, 0)),
            pl.BlockSpec((1, R, F), lambda i: (0, i, 0)),
            pl.BlockSpec((1, R, F), lambda i: (1, i, 0)),
            pl.BlockSpec((1, R, 1), lambda i: (0, i, 0)),
            pl.BlockSpec((1, R, 1), lambda i: (1, i, 0)),
            pl.BlockSpec((F, F), lambda i: (0, 0)),
            pl.BlockSpec((F, F), lambda i: (0, 0)),
            pl.BlockSpec((1, F), lambda i: (0, 0)),
        ],
        out_specs=pl.BlockSpec((R, F), lambda i: (i, 0)),
        out_shape=jax.ShapeDtypeStruct((N_NODES, F), jnp.float32),
    )(x, psum, psum, pcnt, pcnt, wl, wr, bias)


def kernel(x, edge_index, W_l, b_l, W_r, b_r):
    ei_flat = edge_index.astype(jnp.int32).reshape(2 * N_EDGES)
    psum, pcnt = _sc_aggregate(x, ei_flat)
    psum3 = psum.reshape(NC, NPAD, F)
    pcnt3 = pcnt.reshape(NC, NPAD, 1)
    bias = (b_l + b_r).reshape(1, F)
    return _tc_combine(x, psum3, pcnt3, W_l, W_r, bias)
